# baseline (speedup 1.0000x reference)
import numpy as np
import jax
import jax.numpy as jnp
from jax import lax
from jax.experimental import pallas as pl
from jax.experimental.pallas import tpu as pltpu

_ERASE_FRAC_LOWER = 0.05
_ERASE_FRAC_UPPER = 0.1
_ERASE_RATIO = 0.3

_H = 512
_W = 512
_C = 3
_B = 8


def _rotl(x, r):
    return ((x << np.uint32(r)) | (x >> np.uint32(32 - r))).astype(np.uint32)


def _threefry2x32(k0, k1, x0, x1):
    x0 = np.asarray(x0, np.uint32).copy()
    x1 = np.asarray(x1, np.uint32).copy()
    ks = [np.uint32(k0), np.uint32(k1),
          np.uint32(np.uint32(k0) ^ np.uint32(k1) ^ np.uint32(0x1BD11BDA))]
    rotations = [(13, 15, 26, 6), (17, 29, 16, 24)]
    x0 = (x0 + ks[0]).astype(np.uint32)
    x1 = (x1 + ks[1]).astype(np.uint32)
    for i in range(5):
        for r in rotations[i % 2]:
            x0 = (x0 + x1).astype(np.uint32)
            x1 = _rotl(x1, r)
            x1 = (x1 ^ x0).astype(np.uint32)
        x0 = (x0 + ks[(i + 1) % 3]).astype(np.uint32)
        x1 = (x1 + ks[(i + 2) % 3] + np.uint32(i + 1)).astype(np.uint32)
    return x0, x1


def _split(k0, k1, num):
    c2 = np.arange(num, dtype=np.uint32)
    b1, b2 = _threefry2x32(k0, k1, np.zeros(num, np.uint32), c2)
    return np.stack([b1, b2], axis=1)


def _random_bits32(k0, k1):
    b1, b2 = _threefry2x32(k0, k1, np.zeros(1, np.uint32), np.zeros(1, np.uint32))
    return np.uint32(b1[0] ^ b2[0])


def _uniform(k0, k1, minval, maxval):
    bits = _random_bits32(k0, k1)
    fb = np.uint32((bits >> np.uint32(9)) | np.uint32(0x3F800000))
    u = np.array([fb], np.uint32).view(np.float32)[0] - np.float32(1.0)
    mn, mx = np.float32(minval), np.float32(maxval)
    return np.maximum(mn, np.float32(u * (mx - mn) + mn))


def _randint(k0, k1, minval, maxval):
    sub = _split(k0, k1, 2)
    higher = _random_bits32(sub[0, 0], sub[0, 1])
    lower = _random_bits32(sub[1, 0], sub[1, 1])
    span = np.uint32(np.int32(maxval) - np.int32(minval))
    if maxval <= minval:
        span = np.uint32(1)
    mult = np.uint32(np.uint32(65536) % span)
    mult = np.uint32((mult * mult) % span)
    offset = np.uint32(
        (np.uint32(higher % span) * mult + np.uint32(lower % span)) % span)
    return np.int32(np.int32(minval) + np.int32(offset))


def _rect_params_np(n):
    keys = _split(np.uint32(0), np.uint32(42), n)
    area = np.float32(_H * _W)
    out = np.zeros((4, n), np.int32)
    for i in range(n):
        sub = _split(keys[i, 0], keys[i, 1], 4)
        ta = np.float32(
            _uniform(sub[0, 0], sub[0, 1], _ERASE_FRAC_LOWER, _ERASE_FRAC_UPPER)
            * area)
        tr = _uniform(sub[1, 0], sub[1, 1], _ERASE_RATIO, 1.0 / _ERASE_RATIO)
        th = np.int32(np.round(np.float32(np.sqrt(ta)) * tr))
        tw = np.int32(np.round(np.float32(np.sqrt(ta)) / tr))
        valid = (tw < _W) and (th < _H)
        x = _randint(sub[2, 0], sub[2, 1], 0, max(_W - int(tw), 1))
        y = _randint(sub[3, 0], sub[3, 1], 0, max(_H - int(th), 1))
        y1 = y + th if valid else y
        x1 = x + tw if valid else x
        out[:, i] = (y, y1, x, x1)
    return out


def _erase_body(rect_ref, in_ref, out_ref):
    i = pl.program_id(0)
    rows = lax.broadcasted_iota(jnp.int32, (_H, _W), 0)
    cols = lax.broadcasted_iota(jnp.int32, (_H, _W), 1)
    for b in range(_B):
        img = i * _B + b
        y0 = rect_ref[0, img]
        y1 = rect_ref[1, img]
        x0 = rect_ref[2, img]
        x1 = rect_ref[3, img]
        mask = (rows >= y0) & (rows < y1) & (cols >= x0) & (cols < x1)
        vals = in_ref[b].astype(jnp.int32).astype(jnp.uint8)
        out_ref[b] = jnp.where(mask[None], jnp.uint8(0), vals)


def kernel(inputs):
    n = inputs.shape[0]
    xp = jnp.transpose(inputs, (0, 3, 1, 2))
    rects = jnp.asarray(_rect_params_np(n))

    out = pl.pallas_call(
        _erase_body,
        grid_spec=pltpu.PrefetchScalarGridSpec(
            num_scalar_prefetch=1,
            grid=(n // _B,),
            in_specs=[pl.BlockSpec((_B, _C, _H, _W), lambda i, rect: (i, 0, 0, 0))],
            out_specs=pl.BlockSpec((_B, _C, _H, _W), lambda i, rect: (i, 0, 0, 0)),
        ),
        out_shape=jax.ShapeDtypeStruct((n, _C, _H, _W), jnp.uint8),
        compiler_params=pltpu.CompilerParams(vmem_limit_bytes=100 * 1024 * 1024),
    )(rects, xp)
    return jnp.transpose(out, (0, 2, 3, 1))


# speedup vs baseline: 1.0536x; 1.0536x over previous
import numpy as np
import jax
import jax.numpy as jnp
from jax import lax
from jax.experimental import pallas as pl
from jax.experimental.pallas import tpu as pltpu

_ERASE_FRAC_LOWER = 0.05
_ERASE_FRAC_UPPER = 0.1
_ERASE_RATIO = 0.3

_H = 512
_W = 512
_C = 3
_B = 4


def _rotl(x, r):
    return ((x << np.uint32(r)) | (x >> np.uint32(32 - r))).astype(np.uint32)


def _threefry2x32(k0, k1, x0, x1):
    x0 = np.asarray(x0, np.uint32).copy()
    x1 = np.asarray(x1, np.uint32).copy()
    ks = [np.uint32(k0), np.uint32(k1),
          np.uint32(np.uint32(k0) ^ np.uint32(k1) ^ np.uint32(0x1BD11BDA))]
    rotations = [(13, 15, 26, 6), (17, 29, 16, 24)]
    x0 = (x0 + ks[0]).astype(np.uint32)
    x1 = (x1 + ks[1]).astype(np.uint32)
    for i in range(5):
        for r in rotations[i % 2]:
            x0 = (x0 + x1).astype(np.uint32)
            x1 = _rotl(x1, r)
            x1 = (x1 ^ x0).astype(np.uint32)
        x0 = (x0 + ks[(i + 1) % 3]).astype(np.uint32)
        x1 = (x1 + ks[(i + 2) % 3] + np.uint32(i + 1)).astype(np.uint32)
    return x0, x1


def _split(k0, k1, num):
    c2 = np.arange(num, dtype=np.uint32)
    b1, b2 = _threefry2x32(k0, k1, np.zeros(num, np.uint32), c2)
    return np.stack([b1, b2], axis=1)


def _random_bits32(k0, k1):
    b1, b2 = _threefry2x32(k0, k1, np.zeros(1, np.uint32), np.zeros(1, np.uint32))
    return np.uint32(b1[0] ^ b2[0])


def _uniform(k0, k1, minval, maxval):
    bits = _random_bits32(k0, k1)
    fb = np.uint32((bits >> np.uint32(9)) | np.uint32(0x3F800000))
    u = np.array([fb], np.uint32).view(np.float32)[0] - np.float32(1.0)
    mn, mx = np.float32(minval), np.float32(maxval)
    return np.maximum(mn, np.float32(u * (mx - mn) + mn))


def _randint(k0, k1, minval, maxval):
    sub = _split(k0, k1, 2)
    higher = _random_bits32(sub[0, 0], sub[0, 1])
    lower = _random_bits32(sub[1, 0], sub[1, 1])
    span = np.uint32(np.int32(maxval) - np.int32(minval))
    if maxval <= minval:
        span = np.uint32(1)
    mult = np.uint32(np.uint32(65536) % span)
    mult = np.uint32((mult * mult) % span)
    offset = np.uint32(
        (np.uint32(higher % span) * mult + np.uint32(lower % span)) % span)
    return np.int32(np.int32(minval) + np.int32(offset))


def _rect_params_np(n):
    keys = _split(np.uint32(0), np.uint32(42), n)
    area = np.float32(_H * _W)
    out = np.zeros((4, n), np.int32)
    for i in range(n):
        sub = _split(keys[i, 0], keys[i, 1], 4)
        ta = np.float32(
            _uniform(sub[0, 0], sub[0, 1], _ERASE_FRAC_LOWER, _ERASE_FRAC_UPPER)
            * area)
        tr = _uniform(sub[1, 0], sub[1, 1], _ERASE_RATIO, 1.0 / _ERASE_RATIO)
        th = np.int32(np.round(np.float32(np.sqrt(ta)) * tr))
        tw = np.int32(np.round(np.float32(np.sqrt(ta)) / tr))
        valid = (tw < _W) and (th < _H)
        x = _randint(sub[2, 0], sub[2, 1], 0, max(_W - int(tw), 1))
        y = _randint(sub[3, 0], sub[3, 1], 0, max(_H - int(th), 1))
        y1 = y + th if valid else y
        x1 = x + tw if valid else x
        out[:, i] = (y, y1, x, x1)
    return out


def _erase_body(rect_ref, in_ref, out_ref):
    i = pl.program_id(0)
    rows = lax.broadcasted_iota(jnp.int32, (_H, _W), 0)
    cols = lax.broadcasted_iota(jnp.int32, (_H, _W), 1)
    for b in range(_B):
        img = i * _B + b
        y0 = rect_ref[0, img]
        y1 = rect_ref[1, img]
        x0 = rect_ref[2, img]
        x1 = rect_ref[3, img]
        mask = (rows >= y0) & (rows < y1) & (cols >= x0) & (cols < x1)
        vals = in_ref[b].astype(jnp.int32).astype(jnp.uint8)
        out_ref[b] = jnp.where(mask[None], jnp.uint8(0), vals)


def kernel(inputs):
    n = inputs.shape[0]
    xp = jnp.transpose(inputs, (0, 3, 1, 2))
    rects = jnp.asarray(_rect_params_np(n))

    out = pl.pallas_call(
        _erase_body,
        grid_spec=pltpu.PrefetchScalarGridSpec(
            num_scalar_prefetch=1,
            grid=(n // _B,),
            in_specs=[pl.BlockSpec((_B, _C, _H, _W), lambda i, rect: (i, 0, 0, 0))],
            out_specs=pl.BlockSpec((_B, _C, _H, _W), lambda i, rect: (i, 0, 0, 0)),
        ),
        out_shape=jax.ShapeDtypeStruct((n, _C, _H, _W), jnp.uint8),
    )(rects, xp)
    return jnp.transpose(out, (0, 2, 3, 1))
